# baseline (speedup 1.0000x reference)
import functools

import jax
import jax.numpy as jnp
from jax import lax
from jax.experimental import pallas as pl
from jax.experimental.pallas import tpu as pltpu
from jax.experimental.pallas import tpu_sc as plsc

_F = 26
_V = 100000
_H = 64
_B = 16384
_NC = 2
_NS = 16
_NW = _NC * _NS
_CB = _B // _NW
_RPC = 8
_NCH = _CB // _RPC

_GROUPS = ((0, 11), (11, 11), (22, 4))

_C = 33408
_NJ = -(-_V // _C)


def _transpose_body(in_ref, out_ref):
  x = in_ref[0]
  out_ref[:, : _H] = jnp.swapaxes(x[:, : _C // 2], 0, 1)
  out_ref[:, _H:] = jnp.swapaxes(x[:, _C // 2:], 0, 1)


def _relayout_tc(tt, field0, nf):
  return pl.pallas_call(
      _transpose_body,
      grid=(nf, _NJ),
      in_specs=[pl.BlockSpec((1, _H, _C), lambda i, j: (i + field0, 0, j))],
      out_specs=pl.BlockSpec((_C // 2, 128), lambda i, j: (i * _NJ + j, 0)),
      out_shape=jax.ShapeDtypeStruct((nf * _NJ * _C // 2, 128), jnp.float32),
  )(tt)


def _make_sc_gather(nf, with_prev):
  mesh = plsc.VectorSubcoreMesh(core_axis_name="c", subcore_axis_name="s")
  kk = _RPC * nf
  scratch = [
      pltpu.VMEM((_NCH, kk), jnp.int32),
      pltpu.VMEM((kk, _H), jnp.float32),
      pltpu.VMEM((_CB, _H), jnp.float32),
      pltpu.SemaphoreType.DMA,
  ]
  if with_prev:
    scratch.append(pltpu.VMEM((_CB, _H), jnp.float32))

  @functools.partial(
      pl.kernel,
      mesh=mesh,
      out_type=jax.ShapeDtypeStruct((_B, _H), jnp.float32),
      compiler_params=pltpu.CompilerParams(use_tc_tiling_on_sc=False),
      scratch_types=scratch,
  )
  def k(tab_hbm, idx_hbm, *rest):
    if with_prev:
      prev_hbm, out_hbm, idx_v, buf_v, acc_v, sem, prev_v = rest
    else:
      out_hbm, idx_v, buf_v, acc_v, sem = rest
    wid = lax.axis_index("s") * _NC + lax.axis_index("c")
    pltpu.sync_copy(idx_hbm.at[wid], idx_v)
    if with_prev:
      pltpu.sync_copy(prev_hbm.at[pl.ds(wid * _CB, _CB)], prev_v)

    def chunk_body(j, carry):
      pltpu.async_copy(tab_hbm.at[idx_v.at[j]], buf_v, sem).wait()
      for r in range(_RPC):
        for g in range(_H // 16):
          row = j * _RPC + r
          v = buf_v[r * nf, pl.ds(16 * g, 16)]
          for i in range(1, nf):
            v = v + buf_v[r * nf + i, pl.ds(16 * g, 16)]
          if with_prev:
            v = v + prev_v[row, pl.ds(16 * g, 16)]
          acc_v[row, pl.ds(16 * g, 16)] = v
      return carry

    lax.fori_loop(0, _NCH, chunk_body, 0)
    pltpu.sync_copy(acc_v, out_hbm.at[pl.ds(wid * _CB, _CB)])

  return k


_gathers = {
    (nf, bool(i)): _make_sc_gather(nf, with_prev=bool(i))
    for i, (f0, nf) in enumerate(_GROUPS)
}


def _grp_indices(x, field0, nf):
  v = x[:, field0:field0 + nf].astype(jnp.int32)
  i_off = (jnp.arange(nf, dtype=jnp.int32) * _NJ)[None, :]
  j, q = v // _C, v % _C
  idx = ((i_off + j) * (_C // 2) + q % (_C // 2)) * 2 + q // (_C // 2)
  return idx.reshape(_NW, _NCH, _RPC * nf)


def kernel(x, tables):
  tt = jnp.transpose(tables, (0, 2, 1))
  tabs = [
      _relayout_tc(tt, f0, nf).reshape(nf * _NJ * _C, _H)
      for f0, nf in _GROUPS
  ]
  part = None
  for gi, (f0, nf) in enumerate(_GROUPS):
    gather = _gathers[(nf, bool(gi))]
    idx = _grp_indices(x, f0, nf)
    if gi == 0:
      part = gather(tabs[gi], idx)
    else:
      part = gather(tabs[gi], idx, part)
  return part


# speedup vs baseline: 1.0864x; 1.0864x over previous
import functools

import jax
import jax.numpy as jnp
from jax import lax
from jax.experimental import pallas as pl
from jax.experimental.pallas import tpu as pltpu
from jax.experimental.pallas import tpu_sc as plsc

_F = 26
_V = 100000
_H = 64
_B = 16384
_NC = 2
_NS = 16
_NW = _NC * _NS
_CB = _B // _NW
_RPC = 8
_NCH = _CB // _RPC

_GROUPS = ((0, 11), (11, 11), (22, 4))

_C = 25088
_NJ = -(-_V // _C)


def _transpose_body(in_ref, out_ref):
  x = in_ref[0]
  out_ref[:, : _H] = jnp.swapaxes(x[:, : _C // 2], 0, 1)
  out_ref[:, _H:] = jnp.swapaxes(x[:, _C // 2:], 0, 1)


def _relayout_tc(tt, field0, nf):
  return pl.pallas_call(
      _transpose_body,
      grid=(nf, _NJ),
      in_specs=[pl.BlockSpec((1, _H, _C), lambda i, j: (i + field0, 0, j))],
      out_specs=pl.BlockSpec((_C // 2, 128), lambda i, j: (i * _NJ + j, 0)),
      out_shape=jax.ShapeDtypeStruct((nf * _NJ * _C // 2, 128), jnp.float32),
  )(tt)


def _make_sc_gather(nf, with_prev):
  mesh = plsc.VectorSubcoreMesh(core_axis_name="c", subcore_axis_name="s")
  kk = _RPC * nf
  scratch = [
      pltpu.VMEM((_NCH, kk), jnp.int32),
      pltpu.VMEM((kk, _H), jnp.float32),
      pltpu.VMEM((_CB, _H), jnp.float32),
      pltpu.SemaphoreType.DMA,
  ]
  if with_prev:
    scratch.append(pltpu.VMEM((_CB, _H), jnp.float32))

  @functools.partial(
      pl.kernel,
      mesh=mesh,
      out_type=jax.ShapeDtypeStruct((_B, _H), jnp.float32),
      compiler_params=pltpu.CompilerParams(use_tc_tiling_on_sc=False),
      scratch_types=scratch,
  )
  def k(tab_hbm, idx_hbm, *rest):
    if with_prev:
      prev_hbm, out_hbm, idx_v, buf_v, acc_v, sem, prev_v = rest
    else:
      out_hbm, idx_v, buf_v, acc_v, sem = rest
    wid = lax.axis_index("s") * _NC + lax.axis_index("c")
    pltpu.sync_copy(idx_hbm.at[wid], idx_v)
    if with_prev:
      pltpu.sync_copy(prev_hbm.at[pl.ds(wid * _CB, _CB)], prev_v)

    def chunk_body(j, carry):
      pltpu.async_copy(tab_hbm.at[idx_v.at[j]], buf_v, sem).wait()
      for r in range(_RPC):
        for g in range(_H // 16):
          row = j * _RPC + r
          v = buf_v[r * nf, pl.ds(16 * g, 16)]
          for i in range(1, nf):
            v = v + buf_v[r * nf + i, pl.ds(16 * g, 16)]
          if with_prev:
            v = v + prev_v[row, pl.ds(16 * g, 16)]
          acc_v[row, pl.ds(16 * g, 16)] = v
      return carry

    lax.fori_loop(0, _NCH, chunk_body, 0)
    pltpu.sync_copy(acc_v, out_hbm.at[pl.ds(wid * _CB, _CB)])

  return k


_gathers = {
    (nf, bool(i)): _make_sc_gather(nf, with_prev=bool(i))
    for i, (f0, nf) in enumerate(_GROUPS)
}


def _grp_indices(x, field0, nf):
  v = x[:, field0:field0 + nf].astype(jnp.int32)
  i_off = (jnp.arange(nf, dtype=jnp.int32) * _NJ)[None, :]
  j, q = v // _C, v % _C
  idx = ((i_off + j) * (_C // 2) + q % (_C // 2)) * 2 + q // (_C // 2)
  return idx.reshape(_NW, _NCH, _RPC * nf)


def kernel(x, tables):
  tt = jnp.transpose(tables, (0, 2, 1))
  tabs = [
      _relayout_tc(tt, f0, nf).reshape(nf * _NJ * _C, _H)
      for f0, nf in _GROUPS
  ]
  part = None
  for gi, (f0, nf) in enumerate(_GROUPS):
    gather = _gathers[(nf, bool(gi))]
    idx = _grp_indices(x, f0, nf)
    if gi == 0:
      part = gather(tabs[gi], idx)
    else:
      part = gather(tabs[gi], idx, part)
  return part
